# baseline (speedup 1.0000x reference)
import functools

import jax
import jax.numpy as jnp
from jax import lax
from jax.experimental import pallas as pl
from jax.experimental.pallas import tpu as pltpu
from jax.experimental.pallas import tpu_sc as plsc

NC = 2
NS = 16
NW = NC * NS
LANES = 16


def _make_sc_gather(n_nodes, n_edges):
    ept = n_edges // NW
    ch = 6272
    assert ept % ch == 0 and ch % LANES == 0
    mesh = plsc.VectorSubcoreMesh(core_axis_name="c", subcore_axis_name="s")

    def body(h_hbm, esrc_hbm, edst_hbm, ew_hbm, x3_hbm,
             table, sbuf, dbuf, hsb, hdb, ewb):
        c = lax.axis_index("c")
        s = lax.axis_index("s")
        base = (s * NC + c) * ept
        pltpu.sync_copy(h_hbm, table)

        def chunk_body(ci, carry):
            off = base + ci * ch
            pltpu.sync_copy(esrc_hbm.at[pl.ds(off, ch)], sbuf)
            pltpu.sync_copy(edst_hbm.at[pl.ds(off, ch)], dbuf)
            pltpu.sync_copy(ew_hbm.at[pl.ds(off, ch)], ewb)

            def vec_body(k, carry2):
                i0 = k * LANES
                si = sbuf[pl.ds(i0, LANES)]
                di = dbuf[pl.ds(i0, LANES)]
                hsb[pl.ds(i0, LANES)] = plsc.load_gather(table, [si])
                hdb[pl.ds(i0, LANES)] = plsc.load_gather(table, [di])
                return carry2

            lax.fori_loop(0, ch // LANES, vec_body, 0, unroll=4)
            pltpu.sync_copy(hsb, x3_hbm.at[pl.ds(off, ch)])
            pltpu.sync_copy(hdb, x3_hbm.at[pl.ds(n_edges + off, ch)])
            pltpu.sync_copy(ewb, x3_hbm.at[pl.ds(2 * n_edges + off, ch)])
            return carry

        lax.fori_loop(0, ept // ch, chunk_body, 0)

    return pl.kernel(
        body,
        out_type=jax.ShapeDtypeStruct((3 * n_edges,), jnp.float32),
        mesh=mesh,
        compiler_params=pltpu.CompilerParams(needs_layout_passes=False),
        scratch_types=[
            pltpu.VMEM((n_nodes,), jnp.float32),
            pltpu.VMEM((ch,), jnp.int32),
            pltpu.VMEM((ch,), jnp.int32),
            pltpu.VMEM((ch,), jnp.float32),
            pltpu.VMEM((ch,), jnp.float32),
            pltpu.VMEM((ch,), jnp.float32),
        ],
    )


_GELU_C = 0.7978845608028654
_GELU_A = 0.044715
_GELU_K = _GELU_A / (_GELU_C * _GELU_C)


def _mlp_body(w1t_ref, b1_ref, w2_ref, b2_ref, x_ref, out_ref):
    xb = x_ref[...]
    t = jnp.dot(w1t_ref[...], xb,
                preferred_element_type=jnp.float32)
    t = t + b1_ref[...]
    s = t * t
    u = t + _GELU_K * (s * t)
    g = t * (1.0 + jnp.tanh(u))
    o = jnp.dot(w2_ref[...], g,
                preferred_element_type=jnp.float32)
    out_ref[0] = (o + b2_ref[0]) * xb[2:3, :]


def _make_tc_mlp(n_edges, hidden):
    cb = 8192
    assert n_edges % cb == 0
    grid = (n_edges // cb,)
    full = lambda shape: pl.BlockSpec(shape, lambda i: tuple(0 for _ in shape))
    return pl.pallas_call(
        _mlp_body,
        grid=grid,
        in_specs=[full((hidden, 3)),
                  full((hidden, 1)),
                  full((1, hidden)),
                  pl.BlockSpec(memory_space=pltpu.SMEM),
                  pl.BlockSpec((3, cb), lambda i: (0, i)),
                  ],
        out_specs=pl.BlockSpec((1, 1, cb), lambda i: (i, 0, 0)),
        out_shape=jax.ShapeDtypeStruct((n_edges // cb, 1, cb), jnp.float32),
    )


def _make_sc_scatter(n_nodes, rows):
    rpt = rows // NW
    kr = 56
    assert rpt % kr == 0
    nps = n_nodes // NS
    assert nps % LANES == 0
    mesh = plsc.VectorSubcoreMesh(core_axis_name="c", subcore_axis_name="s")

    def body(msg_hbm, edst_hbm, parts_hbm, idxb, valb, accl, acc_sh):
        c = lax.axis_index("c")
        s = lax.axis_index("s")
        wid = s * NC + c

        def zb(k, carry):
            accl[pl.ds(k * LANES, LANES)] = jnp.zeros((LANES,), jnp.float32)
            return carry

        lax.fori_loop(0, nps // LANES, zb, 0, unroll=8)
        pltpu.sync_copy(accl, acc_sh.at[pl.ds(s * nps, nps)])
        plsc.subcore_barrier()

        row0 = wid * rpt

        def chunk_body(ci, carry):
            r0 = row0 + ci * kr
            pltpu.sync_copy(edst_hbm.at[pl.ds(r0, kr), :], idxb)
            pltpu.sync_copy(msg_hbm.at[pl.ds(r0, kr), :], valb)

            def rower(j, carry2):
                pltpu.sync_copy(valb.at[j], acc_sh.at[idxb.at[j]], add=True)
                return carry2

            lax.fori_loop(0, kr, rower, 0)
            return carry

        lax.fori_loop(0, rpt // kr, chunk_body, 0)
        plsc.subcore_barrier()

        pltpu.sync_copy(acc_sh.at[pl.ds(s * nps, nps)], accl)
        pltpu.sync_copy(accl, parts_hbm.at[pl.ds(c * n_nodes + s * nps, nps)])

    return pl.kernel(
        body,
        out_type=jax.ShapeDtypeStruct((NC * n_nodes,), jnp.float32),
        mesh=mesh,
        compiler_params=pltpu.CompilerParams(needs_layout_passes=False),
        scratch_types=[
            pltpu.VMEM((kr, 128), jnp.int32),
            pltpu.VMEM((kr, 128), jnp.float32),
            pltpu.VMEM((nps,), jnp.float32),
            pltpu.VMEM_SHARED((n_nodes,), jnp.float32),
        ],
    )


def _combine_body(softplus, lnp_ref, h_ref, p_ref, deg_ref, out_ref):
    h = h_ref[...]
    agg = (p_ref[0] + p_ref[1]) / deg_ref[...]
    x = h + agg
    mu = x
    var = (x - mu) * (x - mu)
    hn = (x - mu) / jnp.sqrt(var + 1e-6) * lnp_ref[0] + lnp_ref[1]
    out_ref[...] = jax.nn.softplus(hn) if softplus else hn


def _make_tc_combine(nrows, softplus):
    full = pl.BlockSpec((nrows, 128), lambda: (0, 0))
    return pl.pallas_call(
        functools.partial(_combine_body, softplus),
        in_specs=[pl.BlockSpec(memory_space=pltpu.SMEM),
                  full,
                  pl.BlockSpec((2, nrows, 128), lambda: (0, 0, 0)),
                  full],
        out_specs=full,
        out_shape=jax.ShapeDtypeStruct((nrows, 128), jnp.float32),
    )


def kernel(eps_2d, esrc, edst, ew, ndeg, W1, b1, W2, b2, ln_scale, ln_bias):
    n_nodes = eps_2d.shape[0] * eps_2d.shape[1]
    n_edges = esrc.shape[0]
    n_layers, _, hidden = W1.shape
    rows = n_edges // 128
    nrows = n_nodes // 128

    h = eps_2d.reshape((n_nodes,))
    edst2d = edst.reshape((rows, 128))
    deg2d = ndeg.reshape((nrows, 128))

    W1t = jnp.transpose(W1, (0, 2, 1)) * _GELU_C
    b1c = b1[:, :, None] * _GELU_C
    W2r = jnp.transpose(W2, (0, 2, 1)) * (0.5 / _GELU_C)

    sc_gather = _make_sc_gather(n_nodes, n_edges)
    tc_mlp = _make_tc_mlp(n_edges, hidden)
    sc_scatter = _make_sc_scatter(n_nodes, rows)

    for i in range(n_layers):
        x3 = sc_gather(h, esrc, edst, ew)
        msgw = tc_mlp(W1t[i], b1c[i], W2r[i], b2[i],
                      x3.reshape((3, n_edges)))
        parts = sc_scatter(msgw.reshape((rows, 128)), edst2d)
        lnp = jnp.stack([ln_scale[i, 0], ln_bias[i, 0]])
        combine = _make_tc_combine(nrows, softplus=(i == n_layers - 1))
        h2d = combine(lnp, h.reshape((nrows, 128)),
                      parts.reshape((2, nrows, 128)), deg2d)
        h = h2d.reshape((n_nodes,))

    return h.reshape(eps_2d.shape)


# speedup vs baseline: 1.9933x; 1.9933x over previous
import functools

import jax
import jax.numpy as jnp
from jax import lax
from jax.experimental import pallas as pl
from jax.experimental.pallas import tpu as pltpu
from jax.experimental.pallas import tpu_sc as plsc

NC = 2
NS = 16
NW = NC * NS
LANES = 16


def _make_sc_gather(n_nodes, n_edges):
    ept = n_edges // NW
    ch = 6272
    assert ept % ch == 0 and ch % LANES == 0
    mesh = plsc.VectorSubcoreMesh(core_axis_name="c", subcore_axis_name="s")

    def body(h_hbm, esrc_hbm, edst_hbm, hs_hbm, hd_hbm,
             table, sbuf, dbuf, hsb, hdb):
        c = lax.axis_index("c")
        s = lax.axis_index("s")
        base = (s * NC + c) * ept
        pltpu.sync_copy(h_hbm, table)

        def chunk_body(ci, carry):
            off = base + ci * ch
            pltpu.sync_copy(esrc_hbm.at[pl.ds(off, ch)], sbuf)
            pltpu.sync_copy(edst_hbm.at[pl.ds(off, ch)], dbuf)

            def vec_body(k, carry2):
                i0 = k * LANES
                si = sbuf[pl.ds(i0, LANES)]
                di = dbuf[pl.ds(i0, LANES)]
                hsb[pl.ds(i0, LANES)] = plsc.load_gather(table, [si])
                hdb[pl.ds(i0, LANES)] = plsc.load_gather(table, [di])
                return carry2

            lax.fori_loop(0, ch // LANES, vec_body, 0, unroll=4)
            pltpu.sync_copy(hsb, hs_hbm.at[pl.ds(off, ch)])
            pltpu.sync_copy(hdb, hd_hbm.at[pl.ds(off, ch)])
            return carry

        lax.fori_loop(0, ept // ch, chunk_body, 0)

    return pl.kernel(
        body,
        out_type=[jax.ShapeDtypeStruct((n_edges,), jnp.float32),
                  jax.ShapeDtypeStruct((n_edges,), jnp.float32)],
        mesh=mesh,
        compiler_params=pltpu.CompilerParams(needs_layout_passes=False),
        scratch_types=[
            pltpu.VMEM((n_nodes,), jnp.float32),
            pltpu.VMEM((ch,), jnp.int32),
            pltpu.VMEM((ch,), jnp.int32),
            pltpu.VMEM((ch,), jnp.float32),
            pltpu.VMEM((ch,), jnp.float32),
        ],
    )


_GELU_C = 0.7978845608028654
_GELU_A = 0.044715
_GELU_K = _GELU_A / (_GELU_C * _GELU_C)


def _mlp_body(w1t_ref, b1_ref, w2_ref, b2_ref, hs_ref, hd_ref, ew_ref,
              out_ref):
    w = ew_ref[...][None, :]
    xb = jnp.concatenate(
        [hs_ref[...][None, :], hd_ref[...][None, :], w], axis=0)
    t = jnp.dot(w1t_ref[...], xb,
                preferred_element_type=jnp.float32)
    t = t + b1_ref[...]
    s = t * t
    u = t + _GELU_K * (s * t)
    g = t * (1.0 + jnp.tanh(u))
    o = jnp.dot(w2_ref[...], g,
                preferred_element_type=jnp.float32)
    out_ref[...] = (((o + b2_ref[0]) * w))[0]


def _make_tc_mlp(n_edges, hidden):
    cb = 8192
    assert n_edges % cb == 0
    grid = (n_edges // cb,)
    full = lambda shape: pl.BlockSpec(shape, lambda i: tuple(0 for _ in shape))
    flat = pl.BlockSpec((cb,), lambda i: (i,))
    return pl.pallas_call(
        _mlp_body,
        grid=grid,
        in_specs=[full((hidden, 3)),
                  full((hidden, 1)),
                  full((1, hidden)),
                  pl.BlockSpec(memory_space=pltpu.SMEM),
                  flat, flat, flat,
                  ],
        out_specs=flat,
        out_shape=jax.ShapeDtypeStruct((n_edges,), jnp.float32),
    )


def _make_sc_scatter(n_nodes, rows):
    rpt = rows // NW
    kr = 56
    assert rpt % kr == 0
    nps = n_nodes // NS
    assert nps % LANES == 0
    mesh = plsc.VectorSubcoreMesh(core_axis_name="c", subcore_axis_name="s")

    def body(msg_hbm, edst_hbm, parts_hbm, idxb, valb, accl, acc_sh):
        c = lax.axis_index("c")
        s = lax.axis_index("s")
        wid = s * NC + c

        def zb(k, carry):
            accl[pl.ds(k * LANES, LANES)] = jnp.zeros((LANES,), jnp.float32)
            return carry

        lax.fori_loop(0, nps // LANES, zb, 0, unroll=8)
        pltpu.sync_copy(accl, acc_sh.at[pl.ds(s * nps, nps)])
        plsc.subcore_barrier()

        row0 = wid * rpt

        def chunk_body(ci, carry):
            r0 = row0 + ci * kr
            pltpu.sync_copy(edst_hbm.at[pl.ds(r0, kr), :], idxb)
            pltpu.sync_copy(msg_hbm.at[pl.ds(r0, kr), :], valb)

            def rower(j, carry2):
                pltpu.sync_copy(valb.at[j], acc_sh.at[idxb.at[j]], add=True)
                return carry2

            lax.fori_loop(0, kr, rower, 0)
            return carry

        lax.fori_loop(0, rpt // kr, chunk_body, 0)
        plsc.subcore_barrier()

        pltpu.sync_copy(acc_sh.at[pl.ds(s * nps, nps)], accl)
        pltpu.sync_copy(accl, parts_hbm.at[pl.ds(c * n_nodes + s * nps, nps)])

    return pl.kernel(
        body,
        out_type=jax.ShapeDtypeStruct((NC * n_nodes,), jnp.float32),
        mesh=mesh,
        compiler_params=pltpu.CompilerParams(needs_layout_passes=False),
        scratch_types=[
            pltpu.VMEM((kr, 128), jnp.int32),
            pltpu.VMEM((kr, 128), jnp.float32),
            pltpu.VMEM((nps,), jnp.float32),
            pltpu.VMEM_SHARED((n_nodes,), jnp.float32),
        ],
    )


def _combine_body(softplus, lnp_ref, h_ref, p_ref, deg_ref, out_ref):
    h = h_ref[...]
    agg = (p_ref[0] + p_ref[1]) / deg_ref[...]
    x = h + agg
    mu = x
    var = (x - mu) * (x - mu)
    hn = (x - mu) / jnp.sqrt(var + 1e-6) * lnp_ref[0] + lnp_ref[1]
    out_ref[...] = jax.nn.softplus(hn) if softplus else hn


def _make_tc_combine(nrows, softplus):
    full = pl.BlockSpec((nrows, 128), lambda: (0, 0))
    return pl.pallas_call(
        functools.partial(_combine_body, softplus),
        in_specs=[pl.BlockSpec(memory_space=pltpu.SMEM),
                  full,
                  pl.BlockSpec((2, nrows, 128), lambda: (0, 0, 0)),
                  full],
        out_specs=full,
        out_shape=jax.ShapeDtypeStruct((nrows, 128), jnp.float32),
    )


def kernel(eps_2d, esrc, edst, ew, ndeg, W1, b1, W2, b2, ln_scale, ln_bias):
    n_nodes = eps_2d.shape[0] * eps_2d.shape[1]
    n_edges = esrc.shape[0]
    n_layers, _, hidden = W1.shape
    rows = n_edges // 128
    nrows = n_nodes // 128

    h = eps_2d.reshape((n_nodes,))
    edst2d = edst.reshape((rows, 128))
    deg2d = ndeg.reshape((nrows, 128))

    W1t = jnp.transpose(W1, (0, 2, 1)) * _GELU_C
    b1c = b1[:, :, None] * _GELU_C
    W2r = jnp.transpose(W2, (0, 2, 1)) * (0.5 / _GELU_C)

    sc_gather = _make_sc_gather(n_nodes, n_edges)
    tc_mlp = _make_tc_mlp(n_edges, hidden)
    sc_scatter = _make_sc_scatter(n_nodes, rows)

    for i in range(n_layers):
        hs, hd = sc_gather(h, esrc, edst)
        msgw = tc_mlp(W1t[i], b1c[i], W2r[i], b2[i], hs, hd, ew)
        parts = sc_scatter(msgw.reshape((rows, 128)), edst2d)
        lnp = jnp.stack([ln_scale[i, 0], ln_bias[i, 0]])
        combine = _make_tc_combine(nrows, softplus=(i == n_layers - 1))
        h2d = combine(lnp, h.reshape((nrows, 128)),
                      parts.reshape((2, nrows, 128)), deg2d)
        h = h2d.reshape((n_nodes,))

    return h.reshape(eps_2d.shape)


# speedup vs baseline: 2.0943x; 1.0507x over previous
import functools

import jax
import jax.numpy as jnp
from jax import lax
from jax.experimental import pallas as pl
from jax.experimental.pallas import tpu as pltpu
from jax.experimental.pallas import tpu_sc as plsc

NC = 2
NS = 16
NW = NC * NS
LANES = 16


def _make_sc_gather(n_nodes, n_edges):
    ept = n_edges // NW
    ch = 6272
    assert ept % ch == 0 and ch % LANES == 0
    mesh = plsc.VectorSubcoreMesh(core_axis_name="c", subcore_axis_name="s")

    def body(h_hbm, esrc_hbm, edst_hbm, hs_hbm, hd_hbm,
             table, sbuf, dbuf, hsb, hdb):
        c = lax.axis_index("c")
        s = lax.axis_index("s")
        base = (s * NC + c) * ept
        pltpu.sync_copy(h_hbm, table)

        def chunk_body(ci, carry):
            off = base + ci * ch
            pltpu.sync_copy(esrc_hbm.at[pl.ds(off, ch)], sbuf)
            pltpu.sync_copy(edst_hbm.at[pl.ds(off, ch)], dbuf)

            def vec_body(k, carry2):
                i0 = k * LANES
                si = sbuf[pl.ds(i0, LANES)]
                di = dbuf[pl.ds(i0, LANES)]
                hsb[pl.ds(i0, LANES)] = plsc.load_gather(table, [si])
                hdb[pl.ds(i0, LANES)] = plsc.load_gather(table, [di])
                return carry2

            lax.fori_loop(0, ch // LANES, vec_body, 0, unroll=4)
            pltpu.sync_copy(hsb, hs_hbm.at[pl.ds(off, ch)])
            pltpu.sync_copy(hdb, hd_hbm.at[pl.ds(off, ch)])
            return carry

        lax.fori_loop(0, ept // ch, chunk_body, 0)

    return pl.kernel(
        body,
        out_type=[jax.ShapeDtypeStruct((n_edges,), jnp.float32),
                  jax.ShapeDtypeStruct((n_edges,), jnp.float32)],
        mesh=mesh,
        compiler_params=pltpu.CompilerParams(needs_layout_passes=False),
        scratch_types=[
            pltpu.VMEM((n_nodes,), jnp.float32),
            pltpu.VMEM((ch,), jnp.int32),
            pltpu.VMEM((ch,), jnp.int32),
            pltpu.VMEM((ch,), jnp.float32),
            pltpu.VMEM((ch,), jnp.float32),
        ],
    )


_GELU_C = 0.7978845608028654
_GELU_A = 0.044715
_GELU_K = _GELU_A / (_GELU_C * _GELU_C)


def _mlp_body(w1t_ref, b1_ref, w2_ref, b2_ref, hs_ref, hd_ref, ew_ref,
              out_ref):
    w = ew_ref[...][None, :]
    xb = jnp.concatenate(
        [hs_ref[...][None, :], hd_ref[...][None, :], w], axis=0)
    t = jnp.dot(w1t_ref[...], xb,
                preferred_element_type=jnp.float32)
    t = t + b1_ref[...]
    s = t * t
    u = t + _GELU_K * (s * t)
    g = t * (1.0 + jnp.tanh(u))
    o = jnp.dot(w2_ref[...], g,
                preferred_element_type=jnp.float32)
    out_ref[...] = (((o + b2_ref[0]) * w))[0]


def _make_tc_mlp(n_edges, hidden):
    cb = 16384
    assert n_edges % cb == 0
    grid = (n_edges // cb,)
    full = lambda shape: pl.BlockSpec(shape, lambda i: tuple(0 for _ in shape))
    flat = pl.BlockSpec((cb,), lambda i: (i,))
    return pl.pallas_call(
        _mlp_body,
        grid=grid,
        in_specs=[full((hidden, 3)),
                  full((hidden, 1)),
                  full((1, hidden)),
                  pl.BlockSpec(memory_space=pltpu.SMEM),
                  flat, flat, flat,
                  ],
        out_specs=flat,
        out_shape=jax.ShapeDtypeStruct((n_edges,), jnp.float32),
    )


def _make_sc_scatter(n_nodes, rows):
    rpt = rows // NW
    kr = 56
    assert rpt % kr == 0
    nps = n_nodes // NS
    assert nps % LANES == 0
    mesh = plsc.VectorSubcoreMesh(core_axis_name="c", subcore_axis_name="s")

    def body(msg_hbm, edst_hbm, parts_hbm, idxb, valb, accl, acc_sh):
        c = lax.axis_index("c")
        s = lax.axis_index("s")
        wid = s * NC + c

        def zb(k, carry):
            accl[pl.ds(k * LANES, LANES)] = jnp.zeros((LANES,), jnp.float32)
            return carry

        lax.fori_loop(0, nps // LANES, zb, 0, unroll=8)
        pltpu.sync_copy(accl, acc_sh.at[pl.ds(s * nps, nps)])
        plsc.subcore_barrier()

        row0 = wid * rpt

        def chunk_body(ci, carry):
            r0 = row0 + ci * kr
            pltpu.sync_copy(edst_hbm.at[pl.ds(r0, kr), :], idxb)
            pltpu.sync_copy(msg_hbm.at[pl.ds(r0, kr), :], valb)

            def rower(j, carry2):
                pltpu.sync_copy(valb.at[j], acc_sh.at[idxb.at[j]], add=True)
                return carry2

            lax.fori_loop(0, kr, rower, 0)
            return carry

        lax.fori_loop(0, rpt // kr, chunk_body, 0)
        plsc.subcore_barrier()

        pltpu.sync_copy(acc_sh.at[pl.ds(s * nps, nps)], accl)
        pltpu.sync_copy(accl, parts_hbm.at[pl.ds(c * n_nodes + s * nps, nps)])

    return pl.kernel(
        body,
        out_type=jax.ShapeDtypeStruct((NC * n_nodes,), jnp.float32),
        mesh=mesh,
        compiler_params=pltpu.CompilerParams(needs_layout_passes=False),
        scratch_types=[
            pltpu.VMEM((kr, 128), jnp.int32),
            pltpu.VMEM((kr, 128), jnp.float32),
            pltpu.VMEM((nps,), jnp.float32),
            pltpu.VMEM_SHARED((n_nodes,), jnp.float32),
        ],
    )


def _combine_body(softplus, lnp_ref, h_ref, p_ref, deg_ref, out_ref):
    h = h_ref[...]
    agg = (p_ref[0] + p_ref[1]) / deg_ref[...]
    x = h + agg
    mu = x
    var = (x - mu) * (x - mu)
    hn = (x - mu) / jnp.sqrt(var + 1e-6) * lnp_ref[0] + lnp_ref[1]
    out_ref[...] = jax.nn.softplus(hn) if softplus else hn


def _make_tc_combine(nrows, softplus):
    full = pl.BlockSpec((nrows, 128), lambda: (0, 0))
    return pl.pallas_call(
        functools.partial(_combine_body, softplus),
        in_specs=[pl.BlockSpec(memory_space=pltpu.SMEM),
                  full,
                  pl.BlockSpec((2, nrows, 128), lambda: (0, 0, 0)),
                  full],
        out_specs=full,
        out_shape=jax.ShapeDtypeStruct((nrows, 128), jnp.float32),
    )


def kernel(eps_2d, esrc, edst, ew, ndeg, W1, b1, W2, b2, ln_scale, ln_bias):
    n_nodes = eps_2d.shape[0] * eps_2d.shape[1]
    n_edges = esrc.shape[0]
    n_layers, _, hidden = W1.shape
    rows = n_edges // 128
    nrows = n_nodes // 128

    h = eps_2d.reshape((n_nodes,))
    edst2d = edst.reshape((rows, 128))
    deg2d = ndeg.reshape((nrows, 128))

    W1t = jnp.transpose(W1, (0, 2, 1)) * _GELU_C
    b1c = b1[:, :, None] * _GELU_C
    W2r = jnp.transpose(W2, (0, 2, 1)) * (0.5 / _GELU_C)

    sc_gather = _make_sc_gather(n_nodes, n_edges)
    tc_mlp = _make_tc_mlp(n_edges, hidden)
    sc_scatter = _make_sc_scatter(n_nodes, rows)

    for i in range(n_layers):
        hs, hd = sc_gather(h, esrc, edst)
        msgw = tc_mlp(W1t[i], b1c[i], W2r[i], b2[i], hs, hd, ew)
        parts = sc_scatter(msgw.reshape((rows, 128)), edst2d)
        lnp = jnp.stack([ln_scale[i, 0], ln_bias[i, 0]])
        combine = _make_tc_combine(nrows, softplus=(i == n_layers - 1))
        h2d = combine(lnp, h.reshape((nrows, 128)),
                      parts.reshape((2, nrows, 128)), deg2d)
        h = h2d.reshape((n_nodes,))

    return h.reshape(eps_2d.shape)


# speedup vs baseline: 2.2091x; 1.0548x over previous
import functools

import jax
import jax.numpy as jnp
from jax import lax
from jax.experimental import pallas as pl
from jax.experimental.pallas import tpu as pltpu
from jax.experimental.pallas import tpu_sc as plsc

NC = 2
NS = 16
NW = NC * NS
LANES = 16


def _make_sc_gather(n_nodes, n_edges):
    ept = n_edges // NW
    ch = 6272
    assert ept % ch == 0 and ch % LANES == 0
    mesh = plsc.VectorSubcoreMesh(core_axis_name="c", subcore_axis_name="s")

    def body(h_hbm, esrc_hbm, edst_hbm, hs_hbm, hd_hbm,
             table, sbuf, dbuf, hsb, hdb):
        c = lax.axis_index("c")
        s = lax.axis_index("s")
        base = (s * NC + c) * ept
        pltpu.sync_copy(h_hbm, table)

        def chunk_body(ci, carry):
            off = base + ci * ch
            pltpu.sync_copy(esrc_hbm.at[pl.ds(off, ch)], sbuf)
            pltpu.sync_copy(edst_hbm.at[pl.ds(off, ch)], dbuf)

            def vec_body(k, carry2):
                i0 = k * LANES
                si = sbuf[pl.ds(i0, LANES)]
                di = dbuf[pl.ds(i0, LANES)]
                hsb[pl.ds(i0, LANES)] = plsc.load_gather(table, [si])
                hdb[pl.ds(i0, LANES)] = plsc.load_gather(table, [di])
                return carry2

            lax.fori_loop(0, ch // LANES, vec_body, 0, unroll=4)
            pltpu.sync_copy(hsb, hs_hbm.at[pl.ds(off, ch)])
            pltpu.sync_copy(hdb, hd_hbm.at[pl.ds(off, ch)])
            return carry

        lax.fori_loop(0, ept // ch, chunk_body, 0)

    return pl.kernel(
        body,
        out_type=[jax.ShapeDtypeStruct((n_edges,), jnp.float32),
                  jax.ShapeDtypeStruct((n_edges,), jnp.float32)],
        mesh=mesh,
        compiler_params=pltpu.CompilerParams(needs_layout_passes=False),
        scratch_types=[
            pltpu.VMEM((n_nodes,), jnp.float32),
            pltpu.VMEM((ch,), jnp.int32),
            pltpu.VMEM((ch,), jnp.int32),
            pltpu.VMEM((ch,), jnp.float32),
            pltpu.VMEM((ch,), jnp.float32),
        ],
    )


_GELU_C = 0.7978845608028654
_GELU_A = 0.044715
_GELU_K = _GELU_A / (_GELU_C * _GELU_C)


def _mlp_body(w1t_ref, b1_ref, w2_ref, b2_ref, hs_ref, hd_ref, ew_ref,
              out_ref):
    w = ew_ref[...][None, :]
    xb = jnp.concatenate(
        [hs_ref[...][None, :], hd_ref[...][None, :], w], axis=0)
    t = jnp.dot(w1t_ref[...], xb,
                preferred_element_type=jnp.float32)
    t = t + b1_ref[...]
    s = t * t
    u = t + _GELU_K * (s * t)
    g = t * (1.0 + jnp.tanh(u))
    o = jnp.dot(w2_ref[...], g,
                preferred_element_type=jnp.float32)
    out_ref[...] = (((o + b2_ref[0]) * w))[0]


def _make_tc_mlp(n_edges, hidden):
    cb = 16384
    assert n_edges % cb == 0
    grid = (n_edges // cb,)
    full = lambda shape: pl.BlockSpec(shape, lambda i: tuple(0 for _ in shape))
    flat = pl.BlockSpec((cb,), lambda i: (i,))
    return pl.pallas_call(
        _mlp_body,
        grid=grid,
        in_specs=[full((hidden, 3)),
                  full((hidden, 1)),
                  full((1, hidden)),
                  pl.BlockSpec(memory_space=pltpu.SMEM),
                  flat, flat, flat,
                  ],
        out_specs=flat,
        out_shape=jax.ShapeDtypeStruct((n_edges,), jnp.float32),
    )


def _make_sc_scatter(n_nodes, rows):
    rpt = rows // NW
    kr = 56
    assert rpt % kr == 0
    nps = n_nodes // NS
    assert nps % LANES == 0
    mesh = plsc.VectorSubcoreMesh(core_axis_name="c", subcore_axis_name="s")

    def body(msg_hbm, edst_hbm, parts_hbm, idxb, valb, accl, acc_sh, sem):
        c = lax.axis_index("c")
        s = lax.axis_index("s")
        wid = s * NC + c

        def zb(k, carry):
            accl[pl.ds(k * LANES, LANES)] = jnp.zeros((LANES,), jnp.float32)
            return carry

        lax.fori_loop(0, nps // LANES, zb, 0, unroll=8)
        pltpu.sync_copy(accl, acc_sh.at[pl.ds(s * nps, nps)])
        plsc.subcore_barrier()

        row0 = wid * rpt

        def chunk_body(ci, carry):
            r0 = row0 + ci * kr
            pltpu.sync_copy(edst_hbm.at[pl.ds(r0, kr), :], idxb)
            pltpu.sync_copy(msg_hbm.at[pl.ds(r0, kr), :], valb)
            kf = 14
            for j0 in range(0, kr, kf):
                descs = [
                    pltpu.async_copy(valb.at[j0 + j],
                                     acc_sh.at[idxb.at[j0 + j]], sem,
                                     add=True)
                    for j in range(kf)
                ]
                for d in descs:
                    d.wait()
            return carry

        lax.fori_loop(0, rpt // kr, chunk_body, 0)
        plsc.subcore_barrier()

        pltpu.sync_copy(acc_sh.at[pl.ds(s * nps, nps)], accl)
        pltpu.sync_copy(accl, parts_hbm.at[pl.ds(c * n_nodes + s * nps, nps)])

    return pl.kernel(
        body,
        out_type=jax.ShapeDtypeStruct((NC * n_nodes,), jnp.float32),
        mesh=mesh,
        compiler_params=pltpu.CompilerParams(needs_layout_passes=False),
        scratch_types=[
            pltpu.VMEM((kr, 128), jnp.int32),
            pltpu.VMEM((kr, 128), jnp.float32),
            pltpu.VMEM((nps,), jnp.float32),
            pltpu.VMEM_SHARED((n_nodes,), jnp.float32),
            pltpu.SemaphoreType.DMA,
        ],
    )


def _combine_body(softplus, lnp_ref, h_ref, p_ref, deg_ref, out_ref):
    h = h_ref[...]
    agg = (p_ref[0] + p_ref[1]) / deg_ref[...]
    x = h + agg
    mu = x
    var = (x - mu) * (x - mu)
    hn = (x - mu) / jnp.sqrt(var + 1e-6) * lnp_ref[0] + lnp_ref[1]
    out_ref[...] = jax.nn.softplus(hn) if softplus else hn


def _make_tc_combine(nrows, softplus):
    full = pl.BlockSpec((nrows, 128), lambda: (0, 0))
    return pl.pallas_call(
        functools.partial(_combine_body, softplus),
        in_specs=[pl.BlockSpec(memory_space=pltpu.SMEM),
                  full,
                  pl.BlockSpec((2, nrows, 128), lambda: (0, 0, 0)),
                  full],
        out_specs=full,
        out_shape=jax.ShapeDtypeStruct((nrows, 128), jnp.float32),
    )


def kernel(eps_2d, esrc, edst, ew, ndeg, W1, b1, W2, b2, ln_scale, ln_bias):
    n_nodes = eps_2d.shape[0] * eps_2d.shape[1]
    n_edges = esrc.shape[0]
    n_layers, _, hidden = W1.shape
    rows = n_edges // 128
    nrows = n_nodes // 128

    h = eps_2d.reshape((n_nodes,))
    edst2d = edst.reshape((rows, 128))
    deg2d = ndeg.reshape((nrows, 128))

    W1t = jnp.transpose(W1, (0, 2, 1)) * _GELU_C
    b1c = b1[:, :, None] * _GELU_C
    W2r = jnp.transpose(W2, (0, 2, 1)) * (0.5 / _GELU_C)

    sc_gather = _make_sc_gather(n_nodes, n_edges)
    tc_mlp = _make_tc_mlp(n_edges, hidden)
    sc_scatter = _make_sc_scatter(n_nodes, rows)

    for i in range(n_layers):
        hs, hd = sc_gather(h, esrc, edst)
        msgw = tc_mlp(W1t[i], b1c[i], W2r[i], b2[i], hs, hd, ew)
        parts = sc_scatter(msgw.reshape((rows, 128)), edst2d)
        lnp = jnp.stack([ln_scale[i, 0], ln_bias[i, 0]])
        combine = _make_tc_combine(nrows, softplus=(i == n_layers - 1))
        h2d = combine(lnp, h.reshape((nrows, 128)),
                      parts.reshape((2, nrows, 128)), deg2d)
        h = h2d.reshape((n_nodes,))

    return h.reshape(eps_2d.shape)


# speedup vs baseline: 2.2981x; 1.0403x over previous
import functools

import jax
import jax.numpy as jnp
from jax import lax
from jax.experimental import pallas as pl
from jax.experimental.pallas import tpu as pltpu
from jax.experimental.pallas import tpu_sc as plsc

NC = 2
NS = 16
NW = NC * NS
LANES = 16


def _make_sc_gather(n_nodes, n_edges):
    ept = n_edges // NW
    ch = 6272
    assert ept % ch == 0 and ch % LANES == 0
    mesh = plsc.VectorSubcoreMesh(core_axis_name="c", subcore_axis_name="s")

    nch = ept // ch

    def body(h_hbm, esrc_hbm, edst_hbm, hs_hbm, hd_hbm, table,
             sbuf0, dbuf0, hsb0, hdb0, sbuf1, dbuf1, hsb1, hdb1,
             sem_in, sem_out):
        c = lax.axis_index("c")
        s = lax.axis_index("s")
        base = (s * NC + c) * ept
        pltpu.sync_copy(h_hbm, table)

        bufs = [(sbuf0, dbuf0, hsb0, hdb0), (sbuf1, dbuf1, hsb1, hdb1)]

        def gather_chunk(sbuf, dbuf, hsb, hdb):
            def vec_body(k, carry2):
                i0 = k * LANES
                si = sbuf[pl.ds(i0, LANES)]
                di = dbuf[pl.ds(i0, LANES)]
                hsb[pl.ds(i0, LANES)] = plsc.load_gather(table, [si])
                hdb[pl.ds(i0, LANES)] = plsc.load_gather(table, [di])
                return carry2

            lax.fori_loop(0, ch // LANES, vec_body, 0, unroll=4)

        def start_in(ci, bufset):
            off = base + ci * ch
            return (pltpu.async_copy(esrc_hbm.at[pl.ds(off, ch)],
                                     bufset[0], sem_in),
                    pltpu.async_copy(edst_hbm.at[pl.ds(off, ch)],
                                     bufset[1], sem_in))

        in_descs = {0: start_in(0, bufs[0])}
        out_descs = {}
        for ci in range(nch):
            bufset = bufs[ci % 2]
            for d in in_descs.pop(ci):
                d.wait()
            if ci + 1 < nch:
                in_descs[ci + 1] = start_in(ci + 1, bufs[(ci + 1) % 2])
            if ci - 2 in out_descs:
                for d in out_descs.pop(ci - 2):
                    d.wait()
            gather_chunk(*bufset)
            off = base + ci * ch
            out_descs[ci] = (
                pltpu.async_copy(bufset[2], hs_hbm.at[pl.ds(off, ch)],
                                 sem_out),
                pltpu.async_copy(bufset[3], hd_hbm.at[pl.ds(off, ch)],
                                 sem_out))
        for ci in sorted(out_descs):
            for d in out_descs[ci]:
                d.wait()

    return pl.kernel(
        body,
        out_type=[jax.ShapeDtypeStruct((n_edges,), jnp.float32),
                  jax.ShapeDtypeStruct((n_edges,), jnp.float32)],
        mesh=mesh,
        compiler_params=pltpu.CompilerParams(needs_layout_passes=False),
        scratch_types=(
            [pltpu.VMEM((n_nodes,), jnp.float32)]
            + 2 * [pltpu.VMEM((ch,), jnp.int32), pltpu.VMEM((ch,), jnp.int32),
                   pltpu.VMEM((ch,), jnp.float32), pltpu.VMEM((ch,), jnp.float32)]
            + [pltpu.SemaphoreType.DMA, pltpu.SemaphoreType.DMA]
        ),
    )


_GELU_C = 0.7978845608028654
_GELU_A = 0.044715
_GELU_K = _GELU_A / (_GELU_C * _GELU_C)


def _mlp_body(w1t_ref, b1_ref, w2_ref, b2_ref, hs_ref, hd_ref, ew_ref,
              out_ref):
    w = ew_ref[...][None, :]
    xb = jnp.concatenate(
        [hs_ref[...][None, :], hd_ref[...][None, :], w], axis=0)
    t = jnp.dot(w1t_ref[...], xb,
                preferred_element_type=jnp.float32)
    t = t + b1_ref[...]
    s = t * t
    u = t + _GELU_K * (s * t)
    g = t * (1.0 + jnp.tanh(u))
    o = jnp.dot(w2_ref[...], g,
                preferred_element_type=jnp.float32)
    out_ref[...] = (((o + b2_ref[0]) * w))[0]


def _make_tc_mlp(n_edges, hidden):
    cb = 16384
    assert n_edges % cb == 0
    grid = (n_edges // cb,)
    full = lambda shape: pl.BlockSpec(shape, lambda i: tuple(0 for _ in shape))
    flat = pl.BlockSpec((cb,), lambda i: (i,))
    return pl.pallas_call(
        _mlp_body,
        grid=grid,
        in_specs=[full((hidden, 3)),
                  full((hidden, 1)),
                  full((1, hidden)),
                  pl.BlockSpec(memory_space=pltpu.SMEM),
                  flat, flat, flat,
                  ],
        out_specs=flat,
        out_shape=jax.ShapeDtypeStruct((n_edges,), jnp.float32),
    )


def _make_sc_scatter(n_nodes, rows):
    rpt = rows // NW
    kr = 56
    assert rpt % kr == 0
    nps = n_nodes // NS
    assert nps % LANES == 0
    mesh = plsc.VectorSubcoreMesh(core_axis_name="c", subcore_axis_name="s")

    def body(msg_hbm, edst_hbm, parts_hbm, idxb, valb, accl, acc_sh, sem):
        c = lax.axis_index("c")
        s = lax.axis_index("s")
        wid = s * NC + c

        def zb(k, carry):
            accl[pl.ds(k * LANES, LANES)] = jnp.zeros((LANES,), jnp.float32)
            return carry

        lax.fori_loop(0, nps // LANES, zb, 0, unroll=8)
        pltpu.sync_copy(accl, acc_sh.at[pl.ds(s * nps, nps)])
        plsc.subcore_barrier()

        row0 = wid * rpt

        def chunk_body(ci, carry):
            r0 = row0 + ci * kr
            pltpu.sync_copy(edst_hbm.at[pl.ds(r0, kr), :], idxb)
            pltpu.sync_copy(msg_hbm.at[pl.ds(r0, kr), :], valb)
            kf = 14
            for j0 in range(0, kr, kf):
                descs = [
                    pltpu.async_copy(valb.at[j0 + j],
                                     acc_sh.at[idxb.at[j0 + j]], sem,
                                     add=True)
                    for j in range(kf)
                ]
                for d in descs:
                    d.wait()
            return carry

        lax.fori_loop(0, rpt // kr, chunk_body, 0)
        plsc.subcore_barrier()

        pltpu.sync_copy(acc_sh.at[pl.ds(s * nps, nps)], accl)
        pltpu.sync_copy(accl, parts_hbm.at[pl.ds(c * n_nodes + s * nps, nps)])

    return pl.kernel(
        body,
        out_type=jax.ShapeDtypeStruct((NC * n_nodes,), jnp.float32),
        mesh=mesh,
        compiler_params=pltpu.CompilerParams(needs_layout_passes=False),
        scratch_types=[
            pltpu.VMEM((kr, 128), jnp.int32),
            pltpu.VMEM((kr, 128), jnp.float32),
            pltpu.VMEM((nps,), jnp.float32),
            pltpu.VMEM_SHARED((n_nodes,), jnp.float32),
            pltpu.SemaphoreType.DMA,
        ],
    )


def _combine_body(softplus, lnp_ref, h_ref, p_ref, deg_ref, out_ref):
    h = h_ref[...]
    agg = (p_ref[0] + p_ref[1]) / deg_ref[...]
    x = h + agg
    mu = x
    var = (x - mu) * (x - mu)
    hn = (x - mu) / jnp.sqrt(var + 1e-6) * lnp_ref[0] + lnp_ref[1]
    out_ref[...] = jax.nn.softplus(hn) if softplus else hn


def _make_tc_combine(nrows, softplus):
    full = pl.BlockSpec((nrows, 128), lambda: (0, 0))
    return pl.pallas_call(
        functools.partial(_combine_body, softplus),
        in_specs=[pl.BlockSpec(memory_space=pltpu.SMEM),
                  full,
                  pl.BlockSpec((2, nrows, 128), lambda: (0, 0, 0)),
                  full],
        out_specs=full,
        out_shape=jax.ShapeDtypeStruct((nrows, 128), jnp.float32),
    )


def kernel(eps_2d, esrc, edst, ew, ndeg, W1, b1, W2, b2, ln_scale, ln_bias):
    n_nodes = eps_2d.shape[0] * eps_2d.shape[1]
    n_edges = esrc.shape[0]
    n_layers, _, hidden = W1.shape
    rows = n_edges // 128
    nrows = n_nodes // 128

    h = eps_2d.reshape((n_nodes,))
    edst2d = edst.reshape((rows, 128))
    deg2d = ndeg.reshape((nrows, 128))

    W1t = jnp.transpose(W1, (0, 2, 1)) * _GELU_C
    b1c = b1[:, :, None] * _GELU_C
    W2r = jnp.transpose(W2, (0, 2, 1)) * (0.5 / _GELU_C)

    sc_gather = _make_sc_gather(n_nodes, n_edges)
    tc_mlp = _make_tc_mlp(n_edges, hidden)
    sc_scatter = _make_sc_scatter(n_nodes, rows)

    for i in range(n_layers):
        hs, hd = sc_gather(h, esrc, edst)
        msgw = tc_mlp(W1t[i], b1c[i], W2r[i], b2[i], hs, hd, ew)
        parts = sc_scatter(msgw.reshape((rows, 128)), edst2d)
        lnp = jnp.stack([ln_scale[i, 0], ln_bias[i, 0]])
        combine = _make_tc_combine(nrows, softplus=(i == n_layers - 1))
        h2d = combine(lnp, h.reshape((nrows, 128)),
                      parts.reshape((2, nrows, 128)), deg2d)
        h = h2d.reshape((n_nodes,))

    return h.reshape(eps_2d.shape)


# speedup vs baseline: 2.6544x; 1.1550x over previous
import functools

import jax
import jax.numpy as jnp
from jax import lax
from jax.experimental import pallas as pl
from jax.experimental.pallas import tpu as pltpu
from jax.experimental.pallas import tpu_sc as plsc

NC = 2
NS = 16
NW = NC * NS
LANES = 16


def _make_sc_gather(n_nodes, n_edges):
    ept = n_edges // NW
    ch = 6272
    assert ept % ch == 0 and ch % LANES == 0
    mesh = plsc.VectorSubcoreMesh(core_axis_name="c", subcore_axis_name="s")

    nch = ept // ch

    def body(h_hbm, esrc_hbm, edst_hbm, hs_hbm, hd_hbm, table,
             sbuf0, dbuf0, hsb0, hdb0, sbuf1, dbuf1, hsb1, hdb1,
             sem_in, sem_out):
        c = lax.axis_index("c")
        s = lax.axis_index("s")
        base = (s * NC + c) * ept
        pltpu.sync_copy(h_hbm, table)

        bufs = [(sbuf0, dbuf0, hsb0, hdb0), (sbuf1, dbuf1, hsb1, hdb1)]

        def gather_chunk(sbuf, dbuf, hsb, hdb):
            def vec_body(k, carry2):
                i0 = k * LANES
                si = sbuf[pl.ds(i0, LANES)]
                di = dbuf[pl.ds(i0, LANES)]
                hsb[pl.ds(i0, LANES)] = plsc.load_gather(table, [si])
                hdb[pl.ds(i0, LANES)] = plsc.load_gather(table, [di])
                return carry2

            lax.fori_loop(0, ch // LANES, vec_body, 0, unroll=4)

        def start_in(ci, bufset):
            off = base + ci * ch
            return (pltpu.async_copy(esrc_hbm.at[pl.ds(off, ch)],
                                     bufset[0], sem_in),
                    pltpu.async_copy(edst_hbm.at[pl.ds(off, ch)],
                                     bufset[1], sem_in))

        in_descs = {0: start_in(0, bufs[0])}
        out_descs = {}
        for ci in range(nch):
            bufset = bufs[ci % 2]
            for d in in_descs.pop(ci):
                d.wait()
            if ci + 1 < nch:
                in_descs[ci + 1] = start_in(ci + 1, bufs[(ci + 1) % 2])
            if ci - 2 in out_descs:
                for d in out_descs.pop(ci - 2):
                    d.wait()
            gather_chunk(*bufset)
            off = base + ci * ch
            out_descs[ci] = (
                pltpu.async_copy(bufset[2], hs_hbm.at[pl.ds(off, ch)],
                                 sem_out),
                pltpu.async_copy(bufset[3], hd_hbm.at[pl.ds(off, ch)],
                                 sem_out))
        for ci in sorted(out_descs):
            for d in out_descs[ci]:
                d.wait()

    return pl.kernel(
        body,
        out_type=[jax.ShapeDtypeStruct((n_edges,), jnp.float32),
                  jax.ShapeDtypeStruct((n_edges,), jnp.float32)],
        mesh=mesh,
        compiler_params=pltpu.CompilerParams(needs_layout_passes=False),
        scratch_types=(
            [pltpu.VMEM((n_nodes,), jnp.float32)]
            + 2 * [pltpu.VMEM((ch,), jnp.int32), pltpu.VMEM((ch,), jnp.int32),
                   pltpu.VMEM((ch,), jnp.float32), pltpu.VMEM((ch,), jnp.float32)]
            + [pltpu.SemaphoreType.DMA, pltpu.SemaphoreType.DMA]
        ),
    )


_GELU_C = 0.7978845608028654
_GELU_A = 0.044715
_GELU_K = _GELU_A / (_GELU_C * _GELU_C)


def _mlp_body(w1t_ref, w2_ref, b2_ref, hs_ref, hd_ref, ew_ref, out_ref):
    w = ew_ref[...][None, :]
    xb = jnp.concatenate(
        [hs_ref[...][None, :], hd_ref[...][None, :], w,
         jnp.ones_like(w)], axis=0)
    t = jnp.dot(w1t_ref[...], xb,
                preferred_element_type=jnp.float32)
    s = t * t
    u = t + _GELU_K * (s * t)
    g = t * (1.0 + jnp.tanh(u))
    o = jnp.dot(w2_ref[...], g,
                preferred_element_type=jnp.float32)
    out_ref[...] = (((o + b2_ref[0]) * w))[0]


def _make_tc_mlp(n_edges, hidden):
    cb = 16384
    assert n_edges % cb == 0
    grid = (n_edges // cb,)
    full = lambda shape: pl.BlockSpec(shape, lambda i: tuple(0 for _ in shape))
    flat = pl.BlockSpec((cb,), lambda i: (i,))
    return pl.pallas_call(
        _mlp_body,
        grid=grid,
        in_specs=[full((hidden, 4)),
                  full((1, hidden)),
                  pl.BlockSpec(memory_space=pltpu.SMEM),
                  flat, flat, flat,
                  ],
        out_specs=flat,
        out_shape=jax.ShapeDtypeStruct((n_edges,), jnp.float32),
    )


def _make_sc_scatter(n_nodes, rows):
    rpt = rows // NW
    kr = 56
    assert rpt % kr == 0
    nps = n_nodes // NS
    assert nps % LANES == 0
    mesh = plsc.VectorSubcoreMesh(core_axis_name="c", subcore_axis_name="s")

    def body(msg_hbm, edst_hbm, parts_hbm, idxb, valb, accl, acc_sh, sem):
        c = lax.axis_index("c")
        s = lax.axis_index("s")
        wid = s * NC + c

        def zb(k, carry):
            accl[pl.ds(k * LANES, LANES)] = jnp.zeros((LANES,), jnp.float32)
            return carry

        lax.fori_loop(0, nps // LANES, zb, 0, unroll=8)
        pltpu.sync_copy(accl, acc_sh.at[pl.ds(s * nps, nps)])
        plsc.subcore_barrier()

        row0 = wid * rpt

        def chunk_body(ci, carry):
            r0 = row0 + ci * kr
            pltpu.sync_copy(edst_hbm.at[pl.ds(r0, kr), :], idxb)
            pltpu.sync_copy(msg_hbm.at[pl.ds(r0, kr), :], valb)
            kf = 14
            for j0 in range(0, kr, kf):
                descs = [
                    pltpu.async_copy(valb.at[j0 + j],
                                     acc_sh.at[idxb.at[j0 + j]], sem,
                                     add=True)
                    for j in range(kf)
                ]
                for d in descs:
                    d.wait()
            return carry

        lax.fori_loop(0, rpt // kr, chunk_body, 0)
        plsc.subcore_barrier()

        pltpu.sync_copy(acc_sh.at[pl.ds(s * nps, nps)], accl)
        pltpu.sync_copy(accl, parts_hbm.at[pl.ds(c * n_nodes + s * nps, nps)])

    return pl.kernel(
        body,
        out_type=jax.ShapeDtypeStruct((NC * n_nodes,), jnp.float32),
        mesh=mesh,
        compiler_params=pltpu.CompilerParams(needs_layout_passes=False),
        scratch_types=[
            pltpu.VMEM((kr, 128), jnp.int32),
            pltpu.VMEM((kr, 128), jnp.float32),
            pltpu.VMEM((nps,), jnp.float32),
            pltpu.VMEM_SHARED((n_nodes,), jnp.float32),
            pltpu.SemaphoreType.DMA,
        ],
    )


def _combine_body(softplus, lnp_ref, h_ref, p_ref, deg_ref, out_ref):
    h = h_ref[...]
    agg = (p_ref[0] + p_ref[1]) / deg_ref[...]
    x = h + agg
    mu = x
    var = (x - mu) * (x - mu)
    hn = (x - mu) / jnp.sqrt(var + 1e-6) * lnp_ref[0] + lnp_ref[1]
    out_ref[...] = jax.nn.softplus(hn) if softplus else hn


def _make_tc_combine(nrows, softplus):
    full = pl.BlockSpec((nrows, 128), lambda: (0, 0))
    return pl.pallas_call(
        functools.partial(_combine_body, softplus),
        in_specs=[pl.BlockSpec(memory_space=pltpu.SMEM),
                  full,
                  pl.BlockSpec((2, nrows, 128), lambda: (0, 0, 0)),
                  full],
        out_specs=full,
        out_shape=jax.ShapeDtypeStruct((nrows, 128), jnp.float32),
    )


def kernel(eps_2d, esrc, edst, ew, ndeg, W1, b1, W2, b2, ln_scale, ln_bias):
    n_nodes = eps_2d.shape[0] * eps_2d.shape[1]
    n_edges = esrc.shape[0]
    n_layers, _, hidden = W1.shape
    rows = n_edges // 128
    nrows = n_nodes // 128

    h = eps_2d.reshape((n_nodes,))
    edst2d = edst.reshape((rows, 128))
    deg2d = ndeg.reshape((nrows, 128))

    W1t = jnp.concatenate(
        [jnp.transpose(W1, (0, 2, 1)), b1[:, :, None]], axis=2) * _GELU_C
    W2r = jnp.transpose(W2, (0, 2, 1)) * (0.5 / _GELU_C)

    sc_gather = _make_sc_gather(n_nodes, n_edges)
    tc_mlp = _make_tc_mlp(n_edges, hidden)
    sc_scatter = _make_sc_scatter(n_nodes, rows)

    for i in range(n_layers):
        hs, hd = sc_gather(h, esrc, edst)
        msgw = tc_mlp(W1t[i], W2r[i], b2[i], hs, hd, ew)
        parts = sc_scatter(msgw.reshape((rows, 128)), edst2d)
        lnp = jnp.stack([ln_scale[i, 0], ln_bias[i, 0]])
        combine = _make_tc_combine(nrows, softplus=(i == n_layers - 1))
        h2d = combine(lnp, h.reshape((nrows, 128)),
                      parts.reshape((2, nrows, 128)), deg2d)
        h = h2d.reshape((n_nodes,))

    return h.reshape(eps_2d.shape)


# speedup vs baseline: 2.7692x; 1.0432x over previous
import functools

import jax
import jax.numpy as jnp
from jax import lax
from jax.experimental import pallas as pl
from jax.experimental.pallas import tpu as pltpu
from jax.experimental.pallas import tpu_sc as plsc

NC = 2
NS = 16
NW = NC * NS
LANES = 16


def _make_sc_gather(n_nodes, n_edges):
    ept = n_edges // NW
    ch = 6272
    assert ept % ch == 0 and ch % LANES == 0
    mesh = plsc.VectorSubcoreMesh(core_axis_name="c", subcore_axis_name="s")

    nch = ept // ch

    def body(h_hbm, esrc_hbm, edst_hbm, hs_hbm, hd_hbm, table,
             sbuf0, dbuf0, hsb0, hdb0, sbuf1, dbuf1, hsb1, hdb1,
             sem_in, sem_out):
        c = lax.axis_index("c")
        s = lax.axis_index("s")
        base = (s * NC + c) * ept
        pltpu.sync_copy(h_hbm, table)

        bufs = [(sbuf0, dbuf0, hsb0, hdb0), (sbuf1, dbuf1, hsb1, hdb1)]

        def gather_chunk(sbuf, dbuf, hsb, hdb):
            def vec_body(k, carry2):
                i0 = k * LANES
                si = sbuf[pl.ds(i0, LANES)]
                di = dbuf[pl.ds(i0, LANES)]
                hsb[pl.ds(i0, LANES)] = plsc.load_gather(table, [si])
                hdb[pl.ds(i0, LANES)] = plsc.load_gather(table, [di])
                return carry2

            lax.fori_loop(0, ch // LANES, vec_body, 0, unroll=4)

        def start_in(ci, bufset):
            off = base + ci * ch
            return (pltpu.async_copy(esrc_hbm.at[pl.ds(off, ch)],
                                     bufset[0], sem_in),
                    pltpu.async_copy(edst_hbm.at[pl.ds(off, ch)],
                                     bufset[1], sem_in))

        in_descs = {0: start_in(0, bufs[0])}
        out_descs = {}
        for ci in range(nch):
            bufset = bufs[ci % 2]
            for d in in_descs.pop(ci):
                d.wait()
            if ci + 1 < nch:
                in_descs[ci + 1] = start_in(ci + 1, bufs[(ci + 1) % 2])
            if ci - 2 in out_descs:
                for d in out_descs.pop(ci - 2):
                    d.wait()
            gather_chunk(*bufset)
            off = base + ci * ch
            out_descs[ci] = (
                pltpu.async_copy(bufset[2], hs_hbm.at[pl.ds(off, ch)],
                                 sem_out),
                pltpu.async_copy(bufset[3], hd_hbm.at[pl.ds(off, ch)],
                                 sem_out))
        for ci in sorted(out_descs):
            for d in out_descs[ci]:
                d.wait()

    return pl.kernel(
        body,
        out_type=[jax.ShapeDtypeStruct((n_edges,), jnp.float32),
                  jax.ShapeDtypeStruct((n_edges,), jnp.float32)],
        mesh=mesh,
        compiler_params=pltpu.CompilerParams(needs_layout_passes=False),
        scratch_types=(
            [pltpu.VMEM((n_nodes,), jnp.float32)]
            + 2 * [pltpu.VMEM((ch,), jnp.int32), pltpu.VMEM((ch,), jnp.int32),
                   pltpu.VMEM((ch,), jnp.float32), pltpu.VMEM((ch,), jnp.float32)]
            + [pltpu.SemaphoreType.DMA, pltpu.SemaphoreType.DMA]
        ),
    )


_GELU_C = 0.7978845608028654
_GELU_A = 0.044715
_GELU_K = _GELU_A / (_GELU_C * _GELU_C)


def _mlp_body(w1t_ref, w2_ref, b2_ref, hs_ref, hd_ref, ew_ref, out_ref):
    w = ew_ref[...][None, :]
    xb = jnp.concatenate(
        [hs_ref[...][None, :], hd_ref[...][None, :], w,
         jnp.ones_like(w)], axis=0)
    t = jnp.dot(w1t_ref[...], xb,
                preferred_element_type=jnp.float32)
    s = t * t
    u = t + _GELU_K * (s * t)
    g = t * (1.0 + jnp.tanh(u))
    o = jnp.dot(w2_ref[...], g,
                preferred_element_type=jnp.float32)
    out_ref[...] = (((o + b2_ref[0]) * w))[0]


def _make_tc_mlp(n_edges, hidden):
    cb = 32768
    assert n_edges % cb == 0
    grid = (n_edges // cb,)
    full = lambda shape: pl.BlockSpec(shape, lambda i: tuple(0 for _ in shape))
    flat = pl.BlockSpec((cb,), lambda i: (i,))
    return pl.pallas_call(
        _mlp_body,
        grid=grid,
        in_specs=[full((hidden, 4)),
                  full((1, hidden)),
                  pl.BlockSpec(memory_space=pltpu.SMEM),
                  flat, flat, flat,
                  ],
        out_specs=flat,
        out_shape=jax.ShapeDtypeStruct((n_edges,), jnp.float32),
    )


def _make_sc_scatter(n_nodes, rows):
    rpt = rows // NW
    kr = 56
    assert rpt % kr == 0
    nps = n_nodes // NS
    assert nps % LANES == 0
    mesh = plsc.VectorSubcoreMesh(core_axis_name="c", subcore_axis_name="s")

    def body(msg_hbm, edst_hbm, parts_hbm, idxb, valb, accl, acc_sh, sem):
        c = lax.axis_index("c")
        s = lax.axis_index("s")
        wid = s * NC + c

        def zb(k, carry):
            accl[pl.ds(k * LANES, LANES)] = jnp.zeros((LANES,), jnp.float32)
            return carry

        lax.fori_loop(0, nps // LANES, zb, 0, unroll=8)
        pltpu.sync_copy(accl, acc_sh.at[pl.ds(s * nps, nps)])
        plsc.subcore_barrier()

        row0 = wid * rpt

        def chunk_body(ci, carry):
            r0 = row0 + ci * kr
            pltpu.sync_copy(edst_hbm.at[pl.ds(r0, kr), :], idxb)
            pltpu.sync_copy(msg_hbm.at[pl.ds(r0, kr), :], valb)
            kf = 14
            for j0 in range(0, kr, kf):
                descs = [
                    pltpu.async_copy(valb.at[j0 + j],
                                     acc_sh.at[idxb.at[j0 + j]], sem,
                                     add=True)
                    for j in range(kf)
                ]
                for d in descs:
                    d.wait()
            return carry

        lax.fori_loop(0, rpt // kr, chunk_body, 0)
        plsc.subcore_barrier()

        pltpu.sync_copy(acc_sh.at[pl.ds(s * nps, nps)], accl)
        pltpu.sync_copy(accl, parts_hbm.at[pl.ds(c * n_nodes + s * nps, nps)])

    return pl.kernel(
        body,
        out_type=jax.ShapeDtypeStruct((NC * n_nodes,), jnp.float32),
        mesh=mesh,
        compiler_params=pltpu.CompilerParams(needs_layout_passes=False),
        scratch_types=[
            pltpu.VMEM((kr, 128), jnp.int32),
            pltpu.VMEM((kr, 128), jnp.float32),
            pltpu.VMEM((nps,), jnp.float32),
            pltpu.VMEM_SHARED((n_nodes,), jnp.float32),
            pltpu.SemaphoreType.DMA,
        ],
    )


def _combine_body(softplus, lnp_ref, h_ref, p_ref, deg_ref, out_ref):
    h = h_ref[...]
    agg = (p_ref[0] + p_ref[1]) / deg_ref[...]
    x = h + agg
    mu = x
    var = (x - mu) * (x - mu)
    hn = (x - mu) / jnp.sqrt(var + 1e-6) * lnp_ref[0] + lnp_ref[1]
    out_ref[...] = jax.nn.softplus(hn) if softplus else hn


def _make_tc_combine(nrows, softplus):
    full = pl.BlockSpec((nrows, 128), lambda: (0, 0))
    return pl.pallas_call(
        functools.partial(_combine_body, softplus),
        in_specs=[pl.BlockSpec(memory_space=pltpu.SMEM),
                  full,
                  pl.BlockSpec((2, nrows, 128), lambda: (0, 0, 0)),
                  full],
        out_specs=full,
        out_shape=jax.ShapeDtypeStruct((nrows, 128), jnp.float32),
    )


def kernel(eps_2d, esrc, edst, ew, ndeg, W1, b1, W2, b2, ln_scale, ln_bias):
    n_nodes = eps_2d.shape[0] * eps_2d.shape[1]
    n_edges = esrc.shape[0]
    n_layers, _, hidden = W1.shape
    rows = n_edges // 128
    nrows = n_nodes // 128

    h = eps_2d.reshape((n_nodes,))
    edst2d = edst.reshape((rows, 128))
    deg2d = ndeg.reshape((nrows, 128))

    W1t = jnp.concatenate(
        [jnp.transpose(W1, (0, 2, 1)), b1[:, :, None]], axis=2) * _GELU_C
    W2r = jnp.transpose(W2, (0, 2, 1)) * (0.5 / _GELU_C)

    sc_gather = _make_sc_gather(n_nodes, n_edges)
    tc_mlp = _make_tc_mlp(n_edges, hidden)
    sc_scatter = _make_sc_scatter(n_nodes, rows)

    for i in range(n_layers):
        hs, hd = sc_gather(h, esrc, edst)
        msgw = tc_mlp(W1t[i], W2r[i], b2[i], hs, hd, ew)
        parts = sc_scatter(msgw.reshape((rows, 128)), edst2d)
        lnp = jnp.stack([ln_scale[i, 0], ln_bias[i, 0]])
        combine = _make_tc_combine(nrows, softplus=(i == n_layers - 1))
        h2d = combine(lnp, h.reshape((nrows, 128)),
                      parts.reshape((2, nrows, 128)), deg2d)
        h = h2d.reshape((n_nodes,))

    return h.reshape(eps_2d.shape)
